# Initial kernel scaffold; baseline (speedup 1.0000x reference)
import jax
import jax.numpy as jnp
from jax.experimental import pallas as pl


def kernel(indices, tables):
    raise NotImplementedError("write your pallas kernel here")


# speedup vs baseline: 13.8965x; 13.8965x over previous
import functools

import jax
import jax.numpy as jnp
from jax import lax
from jax.experimental import pallas as pl
from jax.experimental.pallas import tpu as pltpu
from jax.experimental.pallas import tpu_sc as plsc

_NW = 32
_RPC = 16


def _sc_lookup(idx2, flat_tab, F, V, E, B):
    n_chunks = B // _RPC
    per_w = n_chunks // _NW
    ipc = _RPC * F
    wpc = _RPC * F * E
    mesh = plsc.VectorSubcoreMesh(core_axis_name="c", subcore_axis_name="s")

    @functools.partial(
        pl.kernel,
        mesh=mesh,
        compiler_params=pltpu.CompilerParams(needs_layout_passes=False),
        out_type=jax.ShapeDtypeStruct((n_chunks, wpc), jnp.float32),
        scratch_types=[
            pltpu.VMEM((F * V * E,), jnp.float32),
            pltpu.VMEM((ipc,), jnp.int32),
            pltpu.VMEM((wpc,), jnp.float32),
        ],
    )
    def k(idx_hbm, tab_hbm, out_hbm, tab_v, idx_v, out_v):
        wid = lax.axis_index("s") * 2 + lax.axis_index("c")
        pltpu.sync_copy(tab_hbm, tab_v)
        lanes = lax.iota(jnp.int32, 16)

        def chunk_body(c, carry):
            chunk = wid * per_w + c
            pltpu.sync_copy(idx_hbm.at[chunk], idx_v)
            for f in range(F):
                ids = plsc.load_gather(idx_v, [lanes * F + f])
                src0 = (ids + f * V) * E
                dst0 = lanes * (F * E) + f * E

                def e_body(e, carry2):
                    w = plsc.load_gather(tab_v, [src0 + e])
                    plsc.store_scatter(out_v, [dst0 + e], w)
                    return carry2

                lax.fori_loop(0, E, e_body, 0)
            pltpu.sync_copy(out_v, out_hbm.at[chunk])
            return carry

        lax.fori_loop(0, per_w, chunk_body, 0)

    return k(idx2, flat_tab)


def kernel(indices, tables):
    F, V, E = tables.shape
    B = indices.shape[0]
    flat_tab = tables.reshape(F * V * E)
    idx2 = indices.reshape(B // _RPC, _RPC * F)
    out = _sc_lookup(idx2, flat_tab, F, V, E, B)
    return out.reshape(B, F * E)


# speedup vs baseline: 15.7634x; 1.1343x over previous
import functools

import jax
import jax.numpy as jnp
from jax import lax
from jax.experimental import pallas as pl
from jax.experimental.pallas import tpu as pltpu
from jax.experimental.pallas import tpu_sc as plsc

_NW = 32
_RPC = 16


def _sc_lookup(idx2, flat_tab, F, V, E, B):
    n_chunks = B // _RPC
    per_w = n_chunks // _NW
    ipc = _RPC * F
    row_w = F * E
    wpc = _RPC * row_w
    mesh = plsc.VectorSubcoreMesh(core_axis_name="c", subcore_axis_name="s")

    @functools.partial(
        pl.kernel,
        mesh=mesh,
        compiler_params=pltpu.CompilerParams(needs_layout_passes=False),
        out_type=jax.ShapeDtypeStruct((n_chunks, wpc), jnp.float32),
        scratch_types=[
            pltpu.VMEM((F * V * E,), jnp.float32),
            pltpu.VMEM((ipc,), jnp.int32),
            pltpu.VMEM((ipc,), jnp.int32),
            pltpu.VMEM((wpc,), jnp.float32),
            pltpu.VMEM((wpc,), jnp.float32),
            pltpu.SemaphoreType.DMA,
            pltpu.SemaphoreType.DMA,
        ],
    )
    def k(idx_hbm, tab_hbm, out_hbm, tab_v, idx_v0, idx_v1, out_v0, out_v1,
          sem0, sem1):
        wid = lax.axis_index("s") * 2 + lax.axis_index("c")
        pltpu.sync_copy(tab_hbm, tab_v)
        lanes = lax.iota(jnp.int32, 16)
        idx_bufs = (idx_v0, idx_v1)
        out_bufs = (out_v0, out_v1)
        sems = (sem0, sem1)

        def chunk_body(g, carry):
            for b in range(2):
                chunk = wid * per_w + 2 * g + b
                pltpu.sync_copy(idx_hbm.at[chunk], idx_bufs[b])

                @pl.when(g > 0)
                def _wait_prev():
                    pltpu.make_async_copy(
                        out_bufs[b], out_hbm.at[chunk], sems[b]
                    ).wait()

                def f_body(f, carry2, b=b):
                    ids = plsc.load_gather(idx_bufs[b], [lanes * F + f])
                    src0 = (ids + f * V) * E
                    dst0 = lanes * row_w + f * E
                    for e in range(E):
                        w = plsc.load_gather(tab_v, [src0 + e])
                        plsc.store_scatter(out_bufs[b], [dst0 + e], w)
                    return carry2

                lax.fori_loop(0, F, f_body, 0)
                pltpu.async_copy(out_bufs[b], out_hbm.at[chunk], sems[b])
            return carry

        lax.fori_loop(0, per_w // 2, chunk_body, 0)
        for b in range(2):
            last = wid * per_w + per_w - 2 + b
            pltpu.make_async_copy(
                out_bufs[b], out_hbm.at[last], sems[b]
            ).wait()

    return k(idx2, flat_tab)


def kernel(indices, tables):
    F, V, E = tables.shape
    B = indices.shape[0]
    flat_tab = tables.reshape(F * V * E)
    idx2 = indices.reshape(B // _RPC, _RPC * F)
    out = _sc_lookup(idx2, flat_tab, F, V, E, B)
    return out.reshape(B, F * E)
